# Initial kernel scaffold; baseline (speedup 1.0000x reference)
import functools

import jax
import jax.numpy as jnp
from jax import lax
from jax.experimental import pallas as pl
from jax.experimental.pallas import tpu as pltpu
from jax.experimental.pallas import tpu_sc as plsc

N_NODES = 10000
N_EDGES = 163840
HIDDEN = 256
HALF = HIDDEN // 2
N_LAYERS = 3

NUM_CORES = 2
NUM_SUBCORES = 16
LANES = 16
CHUNK = 128
EDGES_PER_SUBCORE = N_EDGES // NUM_SUBCORES
NUM_CHUNKS = EDGES_PER_SUBCORE // CHUNK
ROWS_PER_SUBCORE = N_NODES // NUM_SUBCORES
ZROWS = 125


def _sc_propagate(item_flat, src, dst, w16):
    mesh = plsc.VectorSubcoreMesh(core_axis_name="c", subcore_axis_name="s")
    out_t = jax.ShapeDtypeStruct((NUM_CORES * N_NODES, HALF), jnp.float32)

    @functools.partial(
        pl.kernel,
        out_type=[out_t, out_t, out_t],
        mesh=mesh,
        scratch_types=[
            pltpu.VMEM((CHUNK,), jnp.int32),
            pltpu.VMEM((CHUNK,), jnp.int32),
            pltpu.VMEM((CHUNK, LANES), jnp.float32),
            pltpu.VMEM((CHUNK, HALF), jnp.float32),
            pltpu.VMEM((ZROWS, HALF), jnp.float32),
            pltpu.VMEM_SHARED((N_NODES, HALF), jnp.float32),
        ],
    )
    def run(item_hbm, src_hbm, dst_hbm, w16_hbm, e1_hbm, e2_hbm, e3_hbm,
            srcv, dstv, w16v, rows, zbuf, acc):
        c = lax.axis_index("c")
        s = lax.axis_index("s")
        row_base = c * N_NODES
        edge_base = s * EDGES_PER_SUBCORE
        my_row0 = s * ROWS_PER_SUBCORE

        zeros16 = jnp.zeros((LANES,), jnp.float32)

        @pl.loop(0, ZROWS)
        def _(i):
            for r in range(HALF // LANES):
                zbuf[i, pl.ds(r * LANES, LANES)] = zeros16

        outs = [e1_hbm, e2_hbm, e3_hbm]
        for l in range(N_LAYERS):
            tab = item_hbm if l == 0 else outs[l - 1]

            for k in range(ROWS_PER_SUBCORE // ZROWS):
                pltpu.sync_copy(
                    zbuf, acc.at[pl.ds(my_row0 + k * ZROWS, ZROWS)])
            plsc.subcore_barrier()

            @pl.loop(0, NUM_CHUNKS)
            def _(j):
                base = edge_base + j * CHUNK
                pltpu.sync_copy(src_hbm.at[pl.ds(base, CHUNK)], srcv)
                pltpu.sync_copy(dst_hbm.at[pl.ds(base, CHUNK)], dstv)
                pltpu.sync_copy(w16_hbm.at[pl.ds(base, CHUNK)], w16v)
                for r in range(CHUNK // LANES):
                    sl = pl.ds(r * LANES, LANES)
                    srcv[sl] = srcv[sl] + row_base
                pltpu.sync_copy(tab.at[srcv], rows)

                @pl.loop(0, CHUNK)
                def _(e):
                    w = w16v[e]
                    for r in range(HALF // LANES):
                        sl = pl.ds(r * LANES, LANES)
                        rows[e, sl] = rows[e, sl] * w

                pltpu.sync_copy(rows, acc.at[dstv], add=True)

            plsc.subcore_barrier()
            pltpu.sync_copy(
                acc.at[pl.ds(my_row0, ROWS_PER_SUBCORE)],
                outs[l].at[pl.ds(row_base + my_row0, ROWS_PER_SUBCORE)])
            plsc.subcore_barrier()

    return run(item_flat, src, dst, w16)


def _mean_kernel(a_ref, b_ref, c_ref, d_ref, o_ref):
    o_ref[...] = 0.25 * (a_ref[...] + b_ref[...] + c_ref[...] + d_ref[...])


def _mean4(a, b, c, d):
    n = a.shape[0]
    blk = 2500
    spec = pl.BlockSpec((blk, HALF), lambda i: (i, 0))
    return pl.pallas_call(
        _mean_kernel,
        grid=(n // blk,),
        in_specs=[spec, spec, spec, spec],
        out_specs=spec,
        out_shape=jax.ShapeDtypeStruct((n, HALF), jnp.float32),
    )(a, b, c, d)


def kernel(user_table, item_table, edge_index, edge_weight):
    src = edge_index[0]
    dst = edge_index[1]
    item_flat = jnp.concatenate(
        [item_table[:, :HALF], item_table[:, HALF:]], axis=0)
    w16 = jnp.broadcast_to(edge_weight[:, None], (N_EDGES, LANES))

    e1, e2, e3 = _sc_propagate(item_flat, src, dst, w16)
    m = _mean4(item_flat, e1, e2, e3)
    items_emb = jnp.concatenate([m[:N_NODES], m[N_NODES:]], axis=1)
    return (user_table, items_emb)


# speedup vs baseline: 3.1916x; 3.1916x over previous
import functools

import jax
import jax.numpy as jnp
from jax import lax
from jax.experimental import pallas as pl
from jax.experimental.pallas import tpu as pltpu
from jax.experimental.pallas import tpu_sc as plsc

N_NODES = 10000
N_PAD = 10112
N_EDGES = 163840
HIDDEN = 256
HALF = HIDDEN // 2
N_LAYERS = 3

NUM_CORES = 2
NUM_SUBCORES = 16
LANES = 16
CHUNK = 128
EDGES_PER_SUBCORE = N_EDGES // NUM_SUBCORES
NUM_CHUNKS = EDGES_PER_SUBCORE // CHUNK
ROWS_PER_SUBCORE = N_PAD // NUM_SUBCORES
ZROWS = 128


def _sc_propagate(item_flat, src, dst, w16):
    mesh = plsc.VectorSubcoreMesh(core_axis_name="c", subcore_axis_name="s")
    out_t = jax.ShapeDtypeStruct((NUM_CORES * N_PAD, HALF), jnp.float32)

    @functools.partial(
        pl.kernel,
        out_type=[out_t, out_t, out_t],
        mesh=mesh,
        scratch_types=[
            pltpu.VMEM((CHUNK,), jnp.int32),
            pltpu.VMEM((CHUNK,), jnp.int32),
            pltpu.VMEM((CHUNK, LANES), jnp.float32),
            pltpu.VMEM((CHUNK, HALF), jnp.float32),
            pltpu.VMEM((ZROWS, HALF), jnp.float32),
            pltpu.VMEM_SHARED((N_PAD, HALF), jnp.float32),
        ],
    )
    def run(item_hbm, src_hbm, dst_hbm, w16_hbm, e1_hbm, e2_hbm, e3_hbm,
            srcv, dstv, w16v, rows, zbuf, acc):
        c = lax.axis_index("c")
        s = lax.axis_index("s")
        row_base = c * N_PAD
        edge_base = s * EDGES_PER_SUBCORE
        my_row0 = s * ROWS_PER_SUBCORE

        zeros16 = jnp.zeros((LANES,), jnp.float32)

        @pl.loop(0, ZROWS)
        def _(i):
            for r in range(HALF // LANES):
                zbuf[i, pl.ds(r * LANES, LANES)] = zeros16

        outs = [e1_hbm, e2_hbm, e3_hbm]
        for l in range(N_LAYERS):
            tab = item_hbm if l == 0 else outs[l - 1]

            for k in range(ROWS_PER_SUBCORE // ZROWS):
                pltpu.sync_copy(
                    zbuf, acc.at[pl.ds(my_row0 + k * ZROWS, ZROWS)])
            rem = ROWS_PER_SUBCORE % ZROWS
            if rem:
                pltpu.sync_copy(
                    zbuf.at[pl.ds(0, rem)],
                    acc.at[pl.ds(
                        my_row0 + (ROWS_PER_SUBCORE // ZROWS) * ZROWS, rem)])
            plsc.subcore_barrier()

            @pl.loop(0, NUM_CHUNKS)
            def _(j):
                base = edge_base + j * CHUNK
                pltpu.sync_copy(src_hbm.at[pl.ds(base, CHUNK)], srcv)
                pltpu.sync_copy(dst_hbm.at[pl.ds(base, CHUNK)], dstv)
                pltpu.sync_copy(w16_hbm.at[pl.ds(base, CHUNK)], w16v)
                for r in range(CHUNK // LANES):
                    sl = pl.ds(r * LANES, LANES)
                    srcv[sl] = srcv[sl] + row_base
                pltpu.sync_copy(tab.at[srcv], rows)

                @pl.loop(0, CHUNK)
                def _(e):
                    w = w16v[e]
                    for r in range(HALF // LANES):
                        sl = pl.ds(r * LANES, LANES)
                        rows[e, sl] = rows[e, sl] * w

                pltpu.sync_copy(rows, acc.at[dstv], add=True)

            plsc.subcore_barrier()
            pltpu.sync_copy(
                acc.at[pl.ds(my_row0, ROWS_PER_SUBCORE)],
                outs[l].at[pl.ds(row_base + my_row0, ROWS_PER_SUBCORE)])
            plsc.subcore_barrier()

    return run(item_flat, src, dst, w16)


def _mean_kernel(a_ref, b_ref, c_ref, d_ref, o_ref):
    o_ref[...] = 0.25 * (a_ref[...] + b_ref[...] + c_ref[...] + d_ref[...])


def _mean4(a, b, c, d):
    n = a.shape[0]
    blk = n // 16
    spec = pl.BlockSpec((blk, HALF), lambda i: (i, 0))
    return pl.pallas_call(
        _mean_kernel,
        grid=(n // blk,),
        in_specs=[spec, spec, spec, spec],
        out_specs=spec,
        out_shape=jax.ShapeDtypeStruct((n, HALF), jnp.float32),
    )(a, b, c, d)


def kernel(user_table, item_table, edge_index, edge_weight):
    src = edge_index[0]
    dst = edge_index[1]
    pad = jnp.zeros((N_PAD - N_NODES, HALF), jnp.float32)
    item_flat = jnp.concatenate(
        [item_table[:, :HALF], pad, item_table[:, HALF:], pad], axis=0)
    w16 = jnp.broadcast_to(edge_weight[:, None], (N_EDGES, LANES))

    e1, e2, e3 = _sc_propagate(item_flat, src, dst, w16)
    m = _mean4(item_flat, e1, e2, e3)
    items_emb = jnp.concatenate(
        [m[:N_NODES], m[N_PAD:N_PAD + N_NODES]], axis=1)
    return (user_table, items_emb)


# speedup vs baseline: 6.7960x; 2.1294x over previous
import dataclasses
import functools

import jax
import jax.numpy as jnp
from jax import lax
from jax.experimental import pallas as pl
from jax.experimental.pallas import tpu as pltpu
from jax.experimental.pallas import tpu_sc as plsc

N_NODES = 10000
N_PAD = 10112
N_EDGES = 163840
HIDDEN = 256
HALF = HIDDEN // 2
N_LAYERS = 3

NUM_CORES = 2
NUM_SUBCORES = 16
LANES = 16
CHUNK = 80
EDGES_PER_SUBCORE = N_EDGES // NUM_SUBCORES
NUM_CHUNKS = EDGES_PER_SUBCORE // CHUNK
ROWS_PER_SUBCORE = N_PAD // NUM_SUBCORES


def _sc_propagate(item_flat, sdcat, zrows):
    mesh = plsc.VectorSubcoreMesh(core_axis_name="c", subcore_axis_name="s")
    out_t = jax.ShapeDtypeStruct((NUM_CORES * N_PAD, HALF), jnp.float32)
    cp = pltpu.CompilerParams()
    if "needs_layout_passes" in pltpu.CompilerParams.__dataclass_fields__:
        cp = dataclasses.replace(cp, needs_layout_passes=False)
    J = NUM_CHUNKS
    NB = 4
    NR = 2

    @functools.partial(
        pl.kernel,
        out_type=[out_t, out_t, out_t],
        mesh=mesh,
        compiler_params=cp,
        scratch_types=[
            pltpu.VMEM((NB, 3, CHUNK), jnp.int32),
            pltpu.VMEM((NR, CHUNK, HALF), jnp.float32),
            pltpu.VMEM_SHARED((N_PAD, HALF), jnp.float32),
            pltpu.SemaphoreType.DMA((NB,)),
            pltpu.SemaphoreType.DMA((NR,)),
            pltpu.SemaphoreType.DMA((NR,)),
        ],
    )
    def run(item_hbm, sd_hbm, z_hbm, e1_hbm, e2_hbm, e3_hbm,
            sd, rows, acc, in_sem, g_sem, s_sem):
        c = lax.axis_index("c")
        s = lax.axis_index("s")
        row_base = c * N_PAD
        chunk_base = s * J
        my_row0 = s * ROWS_PER_SUBCORE

        def in_copies(jj, b):
            return (
                pltpu.make_async_copy(
                    sd_hbm.at[chunk_base + jj], sd.at[b], in_sem.at[b]),
            )

        def gather_copy(tab, b, rb):
            return pltpu.make_async_copy(
                tab.at[sd.at[b, 0]], rows.at[rb], g_sem.at[rb])

        def scatter_wait(b, rb):
            pltpu.make_async_copy(
                rows.at[rb], acc.at[sd.at[b, 1]], s_sem.at[rb]).wait()

        def munge(b):
            sdb = sd.at[b, 0]
            for r in range(CHUNK // LANES):
                sl = pl.ds(r * LANES, LANES)
                sdb[sl] = sdb[sl] + row_base

        outs = [e1_hbm, e2_hbm, e3_hbm]
        for l in range(N_LAYERS):
            tab = item_hbm if l == 0 else outs[l - 1]

            pltpu.sync_copy(
                z_hbm, acc.at[pl.ds(my_row0, ROWS_PER_SUBCORE)])
            plsc.subcore_barrier()

            for cp in in_copies(0, 0) + in_copies(1, 1) + in_copies(2, 2):
                cp.start()
            for cp in in_copies(0, 0):
                cp.wait()
            munge(0)
            gather_copy(tab, 0, 0).start()

            @pl.loop(0, J, step=NB)
            def _(j4):
                for b in range(NB):
                    jj = j4 + b
                    rb = b % NR
                    rbn = (b + 1) % NR
                    bn = (b + 1) % NB
                    b3 = (b + 3) % NB

                    gather_copy(tab, b, rb).wait()

                    @pl.when(jj >= 1)
                    def _():
                        scatter_wait(b3, rbn)

                    @pl.when(jj < J - 1)
                    def _():
                        for cp in in_copies(jj + 1, bn):
                            cp.wait()
                        munge(bn)
                        gather_copy(tab, bn, rbn).start()

                    @pl.when(jj < J - 3)
                    def _():
                        for cp in in_copies(jj + 3, b3):
                            cp.start()

                    rows_b = rows.at[rb]
                    wbits_b = sd.at[b, 2]

                    @plsc.parallel_loop(0, CHUNK // LANES)
                    def _(g):
                        goff = pl.multiple_of(g * LANES, LANES)
                        wgrp = plsc.bitcast(
                            wbits_b[pl.ds(goff, LANES)], jnp.float32)
                        for li in range(LANES):
                            w = lax.gather(
                                wgrp, jnp.full((LANES, 1), li, jnp.int32),
                                dimension_numbers=lax.GatherDimensionNumbers(
                                    offset_dims=(), collapsed_slice_dims=(0,),
                                    start_index_map=(0,)),
                                slice_sizes=(1,),
                                mode=lax.GatherScatterMode.PROMISE_IN_BOUNDS)
                            e = goff + li
                            for r in range(HALF // LANES):
                                sl = pl.ds(r * LANES, LANES)
                                rows_b[e, sl] = rows_b[e, sl] * w

                    pltpu.async_copy(
                        rows.at[rb], acc.at[sd.at[b, 1]], s_sem.at[rb],
                        add=True)

            scatter_wait((J - 1) % NB, (J - 1) % NR)

            plsc.subcore_barrier()
            pltpu.sync_copy(
                acc.at[pl.ds(my_row0, ROWS_PER_SUBCORE)],
                outs[l].at[pl.ds(row_base + my_row0, ROWS_PER_SUBCORE)])
            plsc.subcore_barrier()

    return run(item_flat, sdcat, zrows)


def _mean_kernel(a_ref, b_ref, c_ref, d_ref, o_ref):
    o_ref[...] = 0.25 * (a_ref[...] + b_ref[...] + c_ref[...] + d_ref[...])


def _mean4(a, b, c, d):
    n = a.shape[0]
    blk = n // 16
    spec = pl.BlockSpec((blk, HALF), lambda i: (i, 0))
    return pl.pallas_call(
        _mean_kernel,
        grid=(n // blk,),
        in_specs=[spec, spec, spec, spec],
        out_specs=spec,
        out_shape=jax.ShapeDtypeStruct((n, HALF), jnp.float32),
    )(a, b, c, d)


def kernel(user_table, item_table, edge_index, edge_weight):
    total_chunks = N_EDGES // CHUNK
    sdcat = jnp.stack(
        [edge_index[0].reshape(total_chunks, CHUNK),
         edge_index[1].reshape(total_chunks, CHUNK),
         jax.lax.bitcast_convert_type(
             edge_weight, jnp.int32).reshape(total_chunks, CHUNK)], axis=1)
    pad = jnp.zeros((N_PAD - N_NODES, HALF), jnp.float32)
    item_flat = jnp.concatenate(
        [item_table[:, :HALF], pad, item_table[:, HALF:], pad], axis=0)

    zrows = jnp.zeros((ROWS_PER_SUBCORE, HALF), jnp.float32)
    e1, e2, e3 = _sc_propagate(item_flat, sdcat, zrows)
    m = _mean4(item_flat, e1, e2, e3)
    items_emb = jnp.concatenate(
        [m[:N_NODES], m[N_PAD:N_PAD + N_NODES]], axis=1)
    return (user_table, items_emb)


# speedup vs baseline: 7.3022x; 1.0745x over previous
import dataclasses
import functools

import jax
import jax.numpy as jnp
from jax import lax
from jax.experimental import pallas as pl
from jax.experimental.pallas import tpu as pltpu
from jax.experimental.pallas import tpu_sc as plsc

N_NODES = 10000
N_PAD = 10112
N_EDGES = 163840
HIDDEN = 256
HALF = HIDDEN // 2
N_LAYERS = 3

NUM_CORES = 2
NUM_SUBCORES = 16
LANES = 16
CHUNK = 80
EDGES_PER_SUBCORE = N_EDGES // NUM_SUBCORES
NUM_CHUNKS = EDGES_PER_SUBCORE // CHUNK
ROWS_PER_SUBCORE = N_PAD // NUM_SUBCORES


def _sc_propagate(item_flat, sdcat, zrows):
    mesh = plsc.VectorSubcoreMesh(core_axis_name="c", subcore_axis_name="s")
    out_t = jax.ShapeDtypeStruct((NUM_CORES * N_PAD, HALF), jnp.float32)
    cp = pltpu.CompilerParams()
    if "needs_layout_passes" in pltpu.CompilerParams.__dataclass_fields__:
        cp = dataclasses.replace(cp, needs_layout_passes=False)
    J = NUM_CHUNKS
    NB = 4
    NR = 2

    @functools.partial(
        pl.kernel,
        out_type=[out_t, out_t, out_t],
        mesh=mesh,
        compiler_params=cp,
        scratch_types=[
            pltpu.VMEM((NB, 3, CHUNK), jnp.int32),
            pltpu.VMEM((NR, CHUNK, HALF), jnp.float32),
            pltpu.VMEM_SHARED((N_PAD, HALF), jnp.float32),
            pltpu.SemaphoreType.DMA((NB,)),
            pltpu.SemaphoreType.DMA((NR,)),
            pltpu.SemaphoreType.DMA((NR,)),
        ],
    )
    def run(item_hbm, sd_hbm, z_hbm, e1_hbm, e2_hbm, e3_hbm,
            sd, rows, acc, in_sem, g_sem, s_sem):
        c = lax.axis_index("c")
        s = lax.axis_index("s")
        row_base = c * N_PAD
        chunk_base = s * J
        my_row0 = s * ROWS_PER_SUBCORE

        def in_copies(jj, b):
            return (
                pltpu.make_async_copy(
                    sd_hbm.at[chunk_base + jj], sd.at[b], in_sem.at[b]),
            )

        def gather_copy(tab, b, rb):
            return pltpu.make_async_copy(
                tab.at[sd.at[b, 0]], rows.at[rb], g_sem.at[rb])

        def scatter_wait(b, rb):
            pltpu.make_async_copy(
                rows.at[rb], acc.at[sd.at[b, 1]], s_sem.at[rb]).wait()

        def munge(b):
            sdb = sd.at[b, 0]
            for r in range(CHUNK // LANES):
                sl = pl.ds(r * LANES, LANES)
                sdb[sl] = sdb[sl] + row_base

        outs = [e1_hbm, e2_hbm, e3_hbm]
        for l in range(N_LAYERS):
            tab = item_hbm if l == 0 else outs[l - 1]

            pltpu.sync_copy(
                z_hbm, acc.at[pl.ds(my_row0, ROWS_PER_SUBCORE)])
            plsc.subcore_barrier()

            for cp in in_copies(0, 0) + in_copies(1, 1) + in_copies(2, 2):
                cp.start()
            for cp in in_copies(0, 0):
                cp.wait()
            munge(0)
            gather_copy(tab, 0, 0).start()

            @pl.loop(0, J, step=NB)
            def _(j4):
                for b in range(NB):
                    jj = j4 + b
                    rb = b % NR
                    rbn = (b + 1) % NR
                    bn = (b + 1) % NB
                    b3 = (b + 3) % NB

                    gather_copy(tab, b, rb).wait()

                    @pl.when(jj >= 1)
                    def _():
                        scatter_wait(b3, rbn)

                    @pl.when(jj < J - 1)
                    def _():
                        for cp in in_copies(jj + 1, bn):
                            cp.wait()
                        munge(bn)
                        gather_copy(tab, bn, rbn).start()

                    @pl.when(jj < J - 3)
                    def _():
                        for cp in in_copies(jj + 3, b3):
                            cp.start()

                    rows_b = rows.at[rb]
                    wbits_b = sd.at[b, 2]

                    @plsc.parallel_loop(0, CHUNK // LANES, unroll=2)
                    def _(g):
                        goff = pl.multiple_of(g * LANES, LANES)
                        wgrp = plsc.bitcast(
                            wbits_b[pl.ds(goff, LANES)], jnp.float32)
                        for li in range(LANES):
                            w = lax.gather(
                                wgrp, jnp.full((LANES, 1), li, jnp.int32),
                                dimension_numbers=lax.GatherDimensionNumbers(
                                    offset_dims=(), collapsed_slice_dims=(0,),
                                    start_index_map=(0,)),
                                slice_sizes=(1,),
                                mode=lax.GatherScatterMode.PROMISE_IN_BOUNDS)
                            e = goff + li
                            for r in range(HALF // LANES):
                                sl = pl.ds(r * LANES, LANES)
                                rows_b[e, sl] = rows_b[e, sl] * w

                    pltpu.async_copy(
                        rows.at[rb], acc.at[sd.at[b, 1]], s_sem.at[rb],
                        add=True)

            scatter_wait((J - 1) % NB, (J - 1) % NR)

            plsc.subcore_barrier()
            pltpu.sync_copy(
                acc.at[pl.ds(my_row0, ROWS_PER_SUBCORE)],
                outs[l].at[pl.ds(row_base + my_row0, ROWS_PER_SUBCORE)])
            plsc.subcore_barrier()

    return run(item_flat, sdcat, zrows)


def _mean_kernel(a_ref, b_ref, c_ref, d_ref, o_ref):
    o_ref[...] = 0.25 * (a_ref[...] + b_ref[...] + c_ref[...] + d_ref[...])


def _mean4(a, b, c, d):
    n = a.shape[0]
    blk = n // 16
    spec = pl.BlockSpec((blk, HALF), lambda i: (i, 0))
    return pl.pallas_call(
        _mean_kernel,
        grid=(n // blk,),
        in_specs=[spec, spec, spec, spec],
        out_specs=spec,
        out_shape=jax.ShapeDtypeStruct((n, HALF), jnp.float32),
    )(a, b, c, d)


def kernel(user_table, item_table, edge_index, edge_weight):
    total_chunks = N_EDGES // CHUNK
    sdcat = jnp.stack(
        [edge_index[0].reshape(total_chunks, CHUNK),
         edge_index[1].reshape(total_chunks, CHUNK),
         jax.lax.bitcast_convert_type(
             edge_weight, jnp.int32).reshape(total_chunks, CHUNK)], axis=1)
    pad = jnp.zeros((N_PAD - N_NODES, HALF), jnp.float32)
    item_flat = jnp.concatenate(
        [item_table[:, :HALF], pad, item_table[:, HALF:], pad], axis=0)

    zrows = jnp.zeros((ROWS_PER_SUBCORE, HALF), jnp.float32)
    e1, e2, e3 = _sc_propagate(item_flat, sdcat, zrows)
    m = _mean4(item_flat, e1, e2, e3)
    items_emb = jnp.concatenate(
        [m[:N_NODES], m[N_PAD:N_PAD + N_NODES]], axis=1)
    return (user_table, items_emb)


# speedup vs baseline: 8.7123x; 1.1931x over previous
import dataclasses
import functools

import jax
import jax.numpy as jnp
from jax import lax
from jax.experimental import pallas as pl
from jax.experimental.pallas import tpu as pltpu
from jax.experimental.pallas import tpu_sc as plsc

N_NODES = 10000
N_PAD = 10112
N_EDGES = 163840
HIDDEN = 256
HALF = HIDDEN // 2
N_LAYERS = 3

NUM_CORES = 2
NUM_SUBCORES = 16
LANES = 16
CHUNK = 128
EDGES_PER_SUBCORE = N_EDGES // NUM_SUBCORES
NUM_CHUNKS = EDGES_PER_SUBCORE // CHUNK
ROWS_PER_SUBCORE = N_PAD // NUM_SUBCORES


def _sc_propagate(item_flat, planes, zrows):
    mesh = plsc.VectorSubcoreMesh(core_axis_name="c", subcore_axis_name="s")
    out_t = jax.ShapeDtypeStruct((NUM_CORES * N_PAD, HALF), jnp.float32)
    cp = pltpu.CompilerParams()
    if "needs_layout_passes" in pltpu.CompilerParams.__dataclass_fields__:
        cp = dataclasses.replace(cp, needs_layout_passes=False)
    J = NUM_CHUNKS
    NB = 4
    NR = 2

    @functools.partial(
        pl.kernel,
        out_type=[out_t, out_t, out_t],
        mesh=mesh,
        compiler_params=cp,
        scratch_types=[
            pltpu.VMEM((4 * NB, CHUNK), jnp.int32),
            pltpu.VMEM((NR, CHUNK, HALF), jnp.float32),
            pltpu.VMEM_SHARED((N_PAD, HALF), jnp.float32),
            pltpu.SemaphoreType.DMA((NB,)),
            pltpu.SemaphoreType.DMA((NR,)),
            pltpu.SemaphoreType.DMA((NR,)),
        ],
    )
    def run(item_hbm, pl_hbm, z_hbm, e1_hbm, e2_hbm, e3_hbm,
            sdv, rows, acc, in_sem, g_sem, s_sem):
        c = lax.axis_index("c")
        s = lax.axis_index("s")
        chunk_base = s * J
        my_row0 = s * ROWS_PER_SUBCORE
        row_base = c * N_PAD

        def in_copies(jj, b):
            g = chunk_base + jj
            return (
                pltpu.make_async_copy(
                    pl_hbm.at[c, g], sdv.at[b], in_sem.at[b]),
                pltpu.make_async_copy(
                    pl_hbm.at[2, g], sdv.at[NB + b], in_sem.at[b]),
                pltpu.make_async_copy(
                    pl_hbm.at[3, g], sdv.at[2 * NB + b], in_sem.at[b]),
            )

        def gather_copy(tab, b, rb):
            return pltpu.make_async_copy(
                tab.at[sdv.at[b]], rows.at[rb], g_sem.at[rb])

        def scatter_wait(b, rb):
            pltpu.make_async_copy(
                rows.at[rb], acc.at[sdv.at[NB + b]], s_sem.at[rb]).wait()

        outs = [e1_hbm, e2_hbm, e3_hbm]
        for l in range(N_LAYERS):
            tab = item_hbm if l == 0 else outs[l - 1]

            pltpu.sync_copy(
                z_hbm, acc.at[pl.ds(my_row0, ROWS_PER_SUBCORE)])
            plsc.subcore_barrier()

            for cp_ in in_copies(0, 0) + in_copies(1, 1) + in_copies(2, 2):
                cp_.start()
            for cp_ in in_copies(0, 0):
                cp_.wait()
            gather_copy(tab, 0, 0).start()

            @pl.loop(0, J, step=NB)
            def _(j4):
                for b in range(NB):
                    jj = j4 + b
                    rb = b % NR
                    rbn = (b + 1) % NR
                    bn = (b + 1) % NB
                    b3 = (b + 3) % NB

                    gather_copy(tab, b, rb).wait()

                    @pl.when(jj >= 1)
                    def _():
                        scatter_wait(b3, rbn)

                    @pl.when(jj < J - 1)
                    def _():
                        for cp_ in in_copies(jj + 1, bn):
                            cp_.wait()
                        gather_copy(tab, bn, rbn).start()

                    @pl.when(jj < J - 3)
                    def _():
                        for cp_ in in_copies(jj + 3, b3):
                            cp_.start()

                    rows_b = rows.at[rb]
                    wbits_b = sdv.at[2 * NB + b]

                    @plsc.parallel_loop(0, CHUNK // LANES, unroll=2)
                    def _(g):
                        goff = pl.multiple_of(g * LANES, LANES)
                        wgrp = plsc.bitcast(
                            wbits_b[pl.ds(goff, LANES)], jnp.float32)
                        for li in range(LANES):
                            w = lax.gather(
                                wgrp, jnp.full((LANES, 1), li, jnp.int32),
                                dimension_numbers=lax.GatherDimensionNumbers(
                                    offset_dims=(), collapsed_slice_dims=(0,),
                                    start_index_map=(0,)),
                                slice_sizes=(1,),
                                mode=lax.GatherScatterMode.PROMISE_IN_BOUNDS)
                            e = goff + li
                            for r in range(HALF // LANES):
                                sl = pl.ds(r * LANES, LANES)
                                rows_b[e, sl] = rows_b[e, sl] * w

                    pltpu.async_copy(
                        rows.at[rb], acc.at[sdv.at[NB + b]], s_sem.at[rb],
                        add=True)

            scatter_wait((J - 1) % NB, (J - 1) % NR)

            plsc.subcore_barrier()
            pltpu.sync_copy(
                acc.at[pl.ds(my_row0, ROWS_PER_SUBCORE)],
                outs[l].at[pl.ds(row_base + my_row0, ROWS_PER_SUBCORE)])
            plsc.subcore_barrier()

    return run(item_flat, planes, zrows)


def _mean_kernel(a_ref, b_ref, c_ref, d_ref, o_ref):
    o_ref[...] = 0.25 * (a_ref[...] + b_ref[...] + c_ref[...] + d_ref[...])


def _mean4(a, b, c, d):
    n = a.shape[0]
    blk = n // 16
    spec = pl.BlockSpec((blk, HALF), lambda i: (i, 0))
    return pl.pallas_call(
        _mean_kernel,
        grid=(n // blk,),
        in_specs=[spec, spec, spec, spec],
        out_specs=spec,
        out_shape=jax.ShapeDtypeStruct((n, HALF), jnp.float32),
    )(a, b, c, d)


def kernel(user_table, item_table, edge_index, edge_weight):
    total_chunks = N_EDGES // CHUNK
    src = edge_index[0]
    dst = edge_index[1]
    wbits = jax.lax.bitcast_convert_type(edge_weight, jnp.int32)
    planes = jnp.stack(
        [src.reshape(total_chunks, CHUNK),
         (src + N_PAD).reshape(total_chunks, CHUNK),
         dst.reshape(total_chunks, CHUNK),
         wbits.reshape(total_chunks, CHUNK)], axis=0)
    pad = jnp.zeros((N_PAD - N_NODES, HALF), jnp.float32)
    item_flat = jnp.concatenate(
        [item_table[:, :HALF], pad, item_table[:, HALF:], pad], axis=0)

    zrows = jnp.zeros((ROWS_PER_SUBCORE, HALF), jnp.float32)
    e1, e2, e3 = _sc_propagate(item_flat, planes, zrows)
    m = _mean4(item_flat, e1, e2, e3)
    items_emb = jnp.concatenate(
        [m[:N_NODES], m[N_PAD:N_PAD + N_NODES]], axis=1)
    return (user_table, items_emb)


# speedup vs baseline: 9.0249x; 1.0359x over previous
import dataclasses
import functools

import jax
import jax.numpy as jnp
from jax import lax
from jax.experimental import pallas as pl
from jax.experimental.pallas import tpu as pltpu
from jax.experimental.pallas import tpu_sc as plsc

N_NODES = 10000
N_PAD = 10112
N_EDGES = 163840
HIDDEN = 256
HALF = HIDDEN // 2
N_LAYERS = 3

NUM_CORES = 2
NUM_SUBCORES = 16
LANES = 16
CHUNK = 64
EDGES_PER_SUBCORE = N_EDGES // NUM_SUBCORES
NUM_CHUNKS = EDGES_PER_SUBCORE // CHUNK
ROWS_PER_SUBCORE = N_PAD // NUM_SUBCORES


def _sc_propagate(item_flat, planes, zrows):
    mesh = plsc.VectorSubcoreMesh(core_axis_name="c", subcore_axis_name="s")
    out_t = jax.ShapeDtypeStruct((NUM_CORES * N_PAD, HALF), jnp.float32)
    cp = pltpu.CompilerParams()
    if "needs_layout_passes" in pltpu.CompilerParams.__dataclass_fields__:
        cp = dataclasses.replace(cp, needs_layout_passes=False)
    J = NUM_CHUNKS
    NB = 8
    NR = 4

    @functools.partial(
        pl.kernel,
        out_type=[out_t, out_t, out_t],
        mesh=mesh,
        compiler_params=cp,
        scratch_types=[
            pltpu.VMEM((3 * NB, CHUNK), jnp.int32),
            pltpu.VMEM((NR, CHUNK, HALF), jnp.float32),
            pltpu.VMEM_SHARED((N_PAD, HALF), jnp.float32),
            pltpu.SemaphoreType.DMA((NB,)),
            pltpu.SemaphoreType.DMA((NR,)),
            pltpu.SemaphoreType.DMA((NR,)),
        ],
    )
    def run(item_hbm, pl_hbm, z_hbm, e1_hbm, e2_hbm, e3_hbm,
            sdv, rows, acc, in_sem, g_sem, s_sem):
        c = lax.axis_index("c")
        s = lax.axis_index("s")
        chunk_base = s * J
        my_row0 = s * ROWS_PER_SUBCORE
        row_base = c * N_PAD

        def in_copies(jj, b):
            g = chunk_base + jj
            return (
                pltpu.make_async_copy(
                    pl_hbm.at[c, g], sdv.at[b], in_sem.at[b]),
                pltpu.make_async_copy(
                    pl_hbm.at[2, g], sdv.at[NB + b], in_sem.at[b]),
                pltpu.make_async_copy(
                    pl_hbm.at[3, g], sdv.at[2 * NB + b], in_sem.at[b]),
            )

        def gather_copy(tab, b, rb):
            return pltpu.make_async_copy(
                tab.at[sdv.at[b]], rows.at[rb], g_sem.at[rb])

        def scatter_wait(b, rb):
            pltpu.make_async_copy(
                rows.at[rb], acc.at[sdv.at[NB + b]], s_sem.at[rb]).wait()

        outs = [e1_hbm, e2_hbm, e3_hbm]
        for l in range(N_LAYERS):
            tab = item_hbm if l == 0 else outs[l - 1]

            pltpu.sync_copy(
                z_hbm, acc.at[pl.ds(my_row0, ROWS_PER_SUBCORE)])
            plsc.subcore_barrier()

            for jj0 in range(4):
                for cp_ in in_copies(jj0, jj0):
                    cp_.start()
            for cp_ in in_copies(0, 0):
                cp_.wait()
            gather_copy(tab, 0, 0).start()
            for cp_ in in_copies(1, 1):
                cp_.wait()
            gather_copy(tab, 1, 1).start()

            @pl.loop(0, J, step=NB)
            def _(j8):
                for b in range(NB):
                    jj = j8 + b
                    rb = b % NR
                    rb2 = (b + 2) % NR
                    b2 = (b + 2) % NB
                    b4 = (b + 4) % NB
                    b6 = (b + 6) % NB

                    gather_copy(tab, b, rb).wait()

                    @pl.when(jj >= 2)
                    def _():
                        scatter_wait(b6, rb2)

                    @pl.when(jj < J - 2)
                    def _():
                        for cp_ in in_copies(jj + 2, b2):
                            cp_.wait()
                        gather_copy(tab, b2, rb2).start()

                    @pl.when(jj < J - 4)
                    def _():
                        for cp_ in in_copies(jj + 4, b4):
                            cp_.start()

                    rows_b = rows.at[rb]
                    wbits_b = sdv.at[2 * NB + b]

                    @plsc.parallel_loop(0, CHUNK // LANES, unroll=2)
                    def _(g):
                        goff = pl.multiple_of(g * LANES, LANES)
                        wgrp = plsc.bitcast(
                            wbits_b[pl.ds(goff, LANES)], jnp.float32)
                        for li in range(LANES):
                            w = lax.gather(
                                wgrp, jnp.full((LANES, 1), li, jnp.int32),
                                dimension_numbers=lax.GatherDimensionNumbers(
                                    offset_dims=(), collapsed_slice_dims=(0,),
                                    start_index_map=(0,)),
                                slice_sizes=(1,),
                                mode=lax.GatherScatterMode.PROMISE_IN_BOUNDS)
                            e = goff + li
                            for r in range(HALF // LANES):
                                sl = pl.ds(r * LANES, LANES)
                                rows_b[e, sl] = rows_b[e, sl] * w

                    pltpu.async_copy(
                        rows.at[rb], acc.at[sdv.at[NB + b]], s_sem.at[rb],
                        add=True)

            scatter_wait((J - 2) % NB, (J - 2) % NR)
            scatter_wait((J - 1) % NB, (J - 1) % NR)

            plsc.subcore_barrier()
            pltpu.sync_copy(
                acc.at[pl.ds(my_row0, ROWS_PER_SUBCORE)],
                outs[l].at[pl.ds(row_base + my_row0, ROWS_PER_SUBCORE)])
            plsc.subcore_barrier()

    return run(item_flat, planes, zrows)


def _mean_kernel(a_ref, b_ref, c_ref, d_ref, o_ref):
    o_ref[...] = 0.25 * (a_ref[...] + b_ref[...] + c_ref[...] + d_ref[...])


def _mean4(a, b, c, d):
    n = a.shape[0]
    blk = n // 16
    spec = pl.BlockSpec((blk, HALF), lambda i: (i, 0))
    return pl.pallas_call(
        _mean_kernel,
        grid=(n // blk,),
        in_specs=[spec, spec, spec, spec],
        out_specs=spec,
        out_shape=jax.ShapeDtypeStruct((n, HALF), jnp.float32),
    )(a, b, c, d)


def kernel(user_table, item_table, edge_index, edge_weight):
    total_chunks = N_EDGES // CHUNK
    src = edge_index[0]
    dst = edge_index[1]
    wbits = jax.lax.bitcast_convert_type(edge_weight, jnp.int32)
    planes = jnp.stack(
        [src.reshape(total_chunks, CHUNK),
         (src + N_PAD).reshape(total_chunks, CHUNK),
         dst.reshape(total_chunks, CHUNK),
         wbits.reshape(total_chunks, CHUNK)], axis=0)
    pad = jnp.zeros((N_PAD - N_NODES, HALF), jnp.float32)
    item_flat = jnp.concatenate(
        [item_table[:, :HALF], pad, item_table[:, HALF:], pad], axis=0)

    zrows = jnp.zeros((ROWS_PER_SUBCORE, HALF), jnp.float32)
    e1, e2, e3 = _sc_propagate(item_flat, planes, zrows)
    m = _mean4(item_flat, e1, e2, e3)
    items_emb = jnp.concatenate(
        [m[:N_NODES], m[N_PAD:N_PAD + N_NODES]], axis=1)
    return (user_table, items_emb)


# speedup vs baseline: 9.1238x; 1.0110x over previous
import dataclasses
import functools

import jax
import jax.numpy as jnp
from jax import lax
from jax.experimental import pallas as pl
from jax.experimental.pallas import tpu as pltpu
from jax.experimental.pallas import tpu_sc as plsc

N_NODES = 10000
N_PAD = 10112
N_EDGES = 163840
HIDDEN = 256
HALF = HIDDEN // 2
N_LAYERS = 3

NUM_CORES = 2
NUM_SUBCORES = 16
LANES = 16
CHUNK = 64
EDGES_PER_SUBCORE = N_EDGES // NUM_SUBCORES
NUM_CHUNKS = EDGES_PER_SUBCORE // CHUNK
ROWS_PER_SUBCORE = N_PAD // NUM_SUBCORES


def _sc_propagate(item_flat, planes, zrows):
    mesh = plsc.VectorSubcoreMesh(core_axis_name="c", subcore_axis_name="s")
    out_t = jax.ShapeDtypeStruct((NUM_CORES * N_PAD, HALF), jnp.float32)
    cp = pltpu.CompilerParams()
    if "needs_layout_passes" in pltpu.CompilerParams.__dataclass_fields__:
        cp = dataclasses.replace(cp, needs_layout_passes=False)
    J = NUM_CHUNKS
    NB = 8
    NR = 4

    @functools.partial(
        pl.kernel,
        out_type=[out_t, out_t, out_t],
        mesh=mesh,
        compiler_params=cp,
        scratch_types=[
            pltpu.VMEM((3 * NB, CHUNK), jnp.int32),
            pltpu.VMEM((NR, CHUNK, HALF), jnp.float32),
            pltpu.VMEM_SHARED((N_PAD, HALF), jnp.float32),
            pltpu.SemaphoreType.DMA((NB,)),
            pltpu.SemaphoreType.DMA((NR,)),
            pltpu.SemaphoreType.DMA((NR,)),
        ],
    )
    def run(item_hbm, pl_hbm, z_hbm, e1_hbm, e2_hbm, e3_hbm,
            sdv, rows, acc, in_sem, g_sem, s_sem):
        c = lax.axis_index("c")
        s = lax.axis_index("s")
        chunk_base = s * J
        my_row0 = s * ROWS_PER_SUBCORE
        row_base = c * N_PAD

        def in_copies(jj, b):
            g = chunk_base + jj
            return (
                pltpu.make_async_copy(
                    pl_hbm.at[c, g], sdv.at[b], in_sem.at[b]),
                pltpu.make_async_copy(
                    pl_hbm.at[2, g], sdv.at[NB + b], in_sem.at[b]),
                pltpu.make_async_copy(
                    pl_hbm.at[3, g], sdv.at[2 * NB + b], in_sem.at[b]),
            )

        def gather_copy(tab, b, rb):
            return pltpu.make_async_copy(
                tab.at[sdv.at[b]], rows.at[rb], g_sem.at[rb])

        def scatter_wait(b, rb):
            pltpu.make_async_copy(
                rows.at[rb], acc.at[sdv.at[NB + b]], s_sem.at[rb]).wait()

        outs = [e1_hbm, e2_hbm, e3_hbm]
        for l in range(N_LAYERS):
            tab = item_hbm if l == 0 else outs[l - 1]

            zcp = pltpu.make_async_copy(
                z_hbm.at[pl.ds(my_row0, ROWS_PER_SUBCORE)],
                acc.at[pl.ds(my_row0, ROWS_PER_SUBCORE)], s_sem.at[0])
            zcp.start()

            for jj0 in range(4):
                for cp_ in in_copies(jj0, jj0):
                    cp_.start()
            for cp_ in in_copies(0, 0):
                cp_.wait()
            gather_copy(tab, 0, 0).start()
            for cp_ in in_copies(1, 1):
                cp_.wait()
            gather_copy(tab, 1, 1).start()
            zcp.wait()
            plsc.subcore_barrier()

            @pl.loop(0, J, step=NB)
            def _(j8):
                for b in range(NB):
                    jj = j8 + b
                    rb = b % NR
                    rb2 = (b + 2) % NR
                    b2 = (b + 2) % NB
                    b4 = (b + 4) % NB
                    b6 = (b + 6) % NB

                    gather_copy(tab, b, rb).wait()

                    @pl.when(jj >= 2)
                    def _():
                        scatter_wait(b6, rb2)

                    @pl.when(jj < J - 2)
                    def _():
                        for cp_ in in_copies(jj + 2, b2):
                            cp_.wait()
                        gather_copy(tab, b2, rb2).start()

                    @pl.when(jj < J - 4)
                    def _():
                        for cp_ in in_copies(jj + 4, b4):
                            cp_.start()

                    rows_b = rows.at[rb]
                    wbits_b = sdv.at[2 * NB + b]

                    @plsc.parallel_loop(0, CHUNK // LANES, unroll=2)
                    def _(g):
                        goff = pl.multiple_of(g * LANES, LANES)
                        wgrp = plsc.bitcast(
                            wbits_b[pl.ds(goff, LANES)], jnp.float32)
                        for li in range(LANES):
                            w = lax.gather(
                                wgrp, jnp.full((LANES, 1), li, jnp.int32),
                                dimension_numbers=lax.GatherDimensionNumbers(
                                    offset_dims=(), collapsed_slice_dims=(0,),
                                    start_index_map=(0,)),
                                slice_sizes=(1,),
                                mode=lax.GatherScatterMode.PROMISE_IN_BOUNDS)
                            e = goff + li
                            for r in range(HALF // LANES):
                                sl = pl.ds(r * LANES, LANES)
                                rows_b[e, sl] = rows_b[e, sl] * w

                    pltpu.async_copy(
                        rows.at[rb], acc.at[sdv.at[NB + b]], s_sem.at[rb],
                        add=True)

            scatter_wait((J - 2) % NB, (J - 2) % NR)
            scatter_wait((J - 1) % NB, (J - 1) % NR)

            plsc.subcore_barrier()
            pltpu.sync_copy(
                acc.at[pl.ds(my_row0, ROWS_PER_SUBCORE)],
                outs[l].at[pl.ds(row_base + my_row0, ROWS_PER_SUBCORE)])
            plsc.subcore_barrier()

    return run(item_flat, planes, zrows)


def _mean_kernel(a_ref, b_ref, c_ref, d_ref, o_ref):
    o_ref[...] = 0.25 * (a_ref[...] + b_ref[...] + c_ref[...] + d_ref[...])


def _mean4(a, b, c, d):
    n = a.shape[0]
    blk = n // 16
    spec = pl.BlockSpec((blk, HALF), lambda i: (i, 0))
    return pl.pallas_call(
        _mean_kernel,
        grid=(n // blk,),
        in_specs=[spec, spec, spec, spec],
        out_specs=spec,
        out_shape=jax.ShapeDtypeStruct((n, HALF), jnp.float32),
    )(a, b, c, d)


def kernel(user_table, item_table, edge_index, edge_weight):
    total_chunks = N_EDGES // CHUNK
    src = edge_index[0]
    dst = edge_index[1]
    wbits = jax.lax.bitcast_convert_type(edge_weight, jnp.int32)
    planes = jnp.stack(
        [src.reshape(total_chunks, CHUNK),
         (src + N_PAD).reshape(total_chunks, CHUNK),
         dst.reshape(total_chunks, CHUNK),
         wbits.reshape(total_chunks, CHUNK)], axis=0)
    pad = jnp.zeros((N_PAD - N_NODES, HALF), jnp.float32)
    item_flat = jnp.concatenate(
        [item_table[:, :HALF], pad, item_table[:, HALF:], pad], axis=0)

    zrows = jnp.zeros((N_PAD, HALF), jnp.float32)
    e1, e2, e3 = _sc_propagate(item_flat, planes, zrows)
    m = _mean4(item_flat, e1, e2, e3)
    items_emb = jnp.concatenate(
        [m[:N_NODES], m[N_PAD:N_PAD + N_NODES]], axis=1)
    return (user_table, items_emb)
